# baseline (speedup 1.0000x reference)
import functools

import jax
import jax.numpy as jnp
from jax import lax
from jax.experimental import pallas as pl
from jax.experimental.pallas import tpu as pltpu
from jax.experimental.pallas import tpu_sc as plsc

E = 10000
K = 16
D = 128
L = 16
NC, NS = 2, 16
E_PAD = 10240
ECS = E_PAD // NC
DP = D // 2
NPP = DP // NS
CHE = 512
NCH = ECS // CHE
GRP = CHE // L

BM = 10240


def _pack_rows(m):
    lo = lax.bitcast_convert_type(m[:DP].astype(jnp.bfloat16), jnp.uint16)
    hi = lax.bitcast_convert_type(m[DP:].astype(jnp.bfloat16), jnp.uint16)
    word = lo.astype(jnp.uint32) | (hi.astype(jnp.uint32) << 16)
    return lax.bitcast_convert_type(word, jnp.int32)


def _pre_body(x_ref, w_ref, b_ref, at_ref, zt_ref):
    x = x_ref[...]
    w = w_ref[...]
    dn = (((0,), (1,)), ((), ()))
    at = lax.dot_general(w[:D, :], x, dn,
                         preferred_element_type=jnp.float32) + b_ref[...]
    zt = lax.dot_general(w[D:, :], x, dn, preferred_element_type=jnp.float32)
    at_ref[...] = _pack_rows(at)
    zt_ref[...] = _pack_rows(zt)


_pre = pl.pallas_call(
    _pre_body,
    grid=(E_PAD // BM,),
    in_specs=[
        pl.BlockSpec((BM, D), lambda i: (i, 0)),
        pl.BlockSpec((2 * D, D), lambda i: (0, 0)),
        pl.BlockSpec((D, 1), lambda i: (0, 0)),
    ],
    out_specs=[
        pl.BlockSpec((DP, BM), lambda i: (0, i)),
        pl.BlockSpec((DP, BM), lambda i: (0, i)),
    ],
    out_shape=[
        jax.ShapeDtypeStruct((DP, E_PAD), jnp.int32),
        jax.ShapeDtypeStruct((DP, E_PAD), jnp.int32),
    ],
)


def _post_body(st_ref, x_ref, w2_ref, b2_ref, u1_ref, c1_ref, u2_ref, c2_ref, o_ref):
    st = st_ref[...]
    dn = (((0,), (0,)), ((), ()))
    msgs = lax.dot_general(st, w2_ref[...], dn,
                           preferred_element_type=jnp.float32) * (1.0 / K)
    msgs = msgs + b2_ref[...]
    x = x_ref[...]
    u1 = u1_ref[...]
    h2 = (jnp.dot(x, u1[:D, :], preferred_element_type=jnp.float32)
          + jnp.dot(msgs, u1[D:, :], preferred_element_type=jnp.float32)
          + c1_ref[...])
    h2 = jnp.maximum(h2, 0.0)
    o_ref[...] = jnp.dot(h2, u2_ref[...], preferred_element_type=jnp.float32) + c2_ref[...]


_post = pl.pallas_call(
    _post_body,
    grid=(E_PAD // BM,),
    in_specs=[
        pl.BlockSpec((D, BM), lambda i: (0, i)),
        pl.BlockSpec((BM, D), lambda i: (i, 0)),
        pl.BlockSpec((D, D), lambda i: (0, 0)),
        pl.BlockSpec((1, D), lambda i: (0, 0)),
        pl.BlockSpec((2 * D, D), lambda i: (0, 0)),
        pl.BlockSpec((1, D), lambda i: (0, 0)),
        pl.BlockSpec((D, D), lambda i: (0, 0)),
        pl.BlockSpec((1, D), lambda i: (0, 0)),
    ],
    out_specs=pl.BlockSpec((BM, D), lambda i: (i, 0)),
    out_shape=jax.ShapeDtypeStruct((E_PAD, D), jnp.float32),
)


def _sc_body(adjt_hbm, at_hbm, zt_hbm, st_hbm, zt_v, adj0, adj1, a0, a1,
             slo0, slo1, shi0, shi1, insem0, insem1, outsem0, outsem1):
    cid = lax.axis_index("c")
    sid = lax.axis_index("s")
    ebase = cid * ECS
    prow0 = sid * NPP

    for p in range(NPP):
        pltpu.sync_copy(zt_hbm.at[prow0 + p], zt_v.at[pl.ds(p * E_PAD, E_PAD)])

    def in_slices(n):
        lo = ebase + n * CHE
        return (adjt_hbm.at[:, pl.ds(lo, CHE)],
                at_hbm.at[pl.ds(prow0, NPP), pl.ds(lo, CHE)])

    def fetch(n, adj_v, a_v, sem):
        adjs, ats = in_slices(n)
        pltpu.async_copy(adjs, adj_v, sem)
        pltpu.async_copy(ats, a_v, sem)

    def fetch_wait(n, adj_v, a_v, sem):
        adjs, ats = in_slices(n)
        pltpu.make_async_copy(adjs, adj_v, sem).wait()
        pltpu.make_async_copy(ats, a_v, sem).wait()

    def out_slices(n):
        lo = ebase + n * CHE
        return (st_hbm.at[pl.ds(prow0, NPP), pl.ds(lo, CHE)],
                st_hbm.at[pl.ds(DP + prow0, NPP), pl.ds(lo, CHE)])

    def store(n, s_lo, s_hi, sem):
        olo, ohi = out_slices(n)
        pltpu.async_copy(s_lo, olo, sem)
        pltpu.async_copy(s_hi, ohi, sem)

    def store_wait(n, s_lo, s_hi, sem):
        olo, ohi = out_slices(n)
        pltpu.make_async_copy(s_lo, olo, sem).wait()
        pltpu.make_async_copy(s_hi, ohi, sem).wait()

    coff = [jnp.full((L,), p * E_PAD, jnp.int32) for p in range(NPP)]

    def compute(adj_v, a_v, s_lo, s_hi):
        def group(g, carry):
            av = [plsc.bitcast(a_v[p, pl.ds(g * L, L)], jnp.bfloat16)
                  for p in range(NPP)]
            acc = [None] * NPP
            for k in range(K):
                jv = adj_v[k, pl.ds(g * L, L)]
                for p in range(NPP):
                    zp = plsc.load_gather(zt_v, [jv + coff[p]])
                    zb = plsc.bitcast(zp, jnp.bfloat16)
                    t = jnp.maximum(av[p] + zb, jnp.bfloat16(0))
                    acc[p] = t if k == 0 else acc[p] + t
            for p in range(NPP):
                lo, hi = plsc.unpack(acc[p], format=plsc.PackFormat.INTERLEAVED)
                s_lo[p, pl.ds(g * L, L)] = lo
                s_hi[p, pl.ds(g * L, L)] = hi
            return carry

        lax.fori_loop(0, GRP, group, 0)

    fetch(0, adj0, a0, insem0)
    PAIRS = NCH // 2

    def pair_body(p, carry):
        n0 = p * 2
        fetch(n0 + 1, adj1, a1, insem1)
        fetch_wait(n0, adj0, a0, insem0)

        @pl.when(p > 0)
        def _():
            store_wait(n0 - 2, slo0, shi0, outsem0)
        compute(adj0, a0, slo0, shi0)
        store(n0, slo0, shi0, outsem0)

        @pl.when(p < PAIRS - 1)
        def _():
            fetch(n0 + 2, adj0, a0, insem0)
        fetch_wait(n0 + 1, adj1, a1, insem1)

        @pl.when(p > 0)
        def _():
            store_wait(n0 - 1, slo1, shi1, outsem1)
        compute(adj1, a1, slo1, shi1)
        store(n0 + 1, slo1, shi1, outsem1)
        return carry

    lax.fori_loop(0, PAIRS, pair_body, 0)
    store_wait(NCH - 2, slo0, shi0, outsem0)
    store_wait(NCH - 1, slo1, shi1, outsem1)


@functools.cache
def _sc_gather_mean():
    return pl.kernel(
        _sc_body,
        mesh=plsc.VectorSubcoreMesh(core_axis_name="c", subcore_axis_name="s"),
        compiler_params=pltpu.CompilerParams(needs_layout_passes=False),
        out_type=jax.ShapeDtypeStruct((D, E_PAD), jnp.float32),
        scratch_types=[
            pltpu.VMEM((NPP * E_PAD,), jnp.int32),
            pltpu.VMEM((K, CHE), jnp.int32),
            pltpu.VMEM((K, CHE), jnp.int32),
            pltpu.VMEM((NPP, CHE), jnp.int32),
            pltpu.VMEM((NPP, CHE), jnp.int32),
            pltpu.VMEM((NPP, CHE), jnp.float32),
            pltpu.VMEM((NPP, CHE), jnp.float32),
            pltpu.VMEM((NPP, CHE), jnp.float32),
            pltpu.VMEM((NPP, CHE), jnp.float32),
            pltpu.SemaphoreType.DMA,
            pltpu.SemaphoreType.DMA,
            pltpu.SemaphoreType.DMA,
            pltpu.SemaphoreType.DMA,
        ],
    )


def kernel(edge_features, edge_adjacency, msg_W1, msg_b1, msg_W2, msg_b2,
           upd_W1, upd_b1, upd_W2, upd_b2):
    xp = jnp.zeros((E_PAD, D), jnp.float32).at[:E].set(edge_features)
    adj = jnp.zeros((E_PAD, K), jnp.int32).at[:E].set(edge_adjacency.astype(jnp.int32))
    adjt = adj.T
    at, zt = _pre(xp, msg_W1, msg_b1.reshape(D, 1))
    st = _sc_gather_mean()(adjt, at, zt)
    out = _post(st, xp, msg_W2, msg_b2.reshape(1, D), upd_W1, upd_b1.reshape(1, D),
                upd_W2, upd_b2.reshape(1, D))
    return out[:E]


# speedup vs baseline: 1.0284x; 1.0284x over previous
import functools

import jax
import jax.numpy as jnp
from jax import lax
from jax.experimental import pallas as pl
from jax.experimental.pallas import tpu as pltpu
from jax.experimental.pallas import tpu_sc as plsc

E = 10000
K = 16
D = 128
L = 16
NC, NS = 2, 16
E_PAD = 10240
ECS = E_PAD // NC
DP = D // 2
NPP = DP // NS
CHE = 512
NCH = ECS // CHE
GRP = CHE // L

BM = 5120


def _pack_rows(m):
    lo = lax.bitcast_convert_type(m[:DP].astype(jnp.bfloat16), jnp.uint16)
    hi = lax.bitcast_convert_type(m[DP:].astype(jnp.bfloat16), jnp.uint16)
    word = lo.astype(jnp.uint32) | (hi.astype(jnp.uint32) << 16)
    return lax.bitcast_convert_type(word, jnp.int32)


def _pre_body(x_ref, w_ref, b_ref, at_ref, zt_ref):
    x = x_ref[...]
    w = w_ref[...]
    dn = (((0,), (1,)), ((), ()))
    at = lax.dot_general(w[:D, :], x, dn,
                         preferred_element_type=jnp.float32) + b_ref[...]
    zt = lax.dot_general(w[D:, :], x, dn, preferred_element_type=jnp.float32)
    at_ref[...] = _pack_rows(at)
    zt_ref[...] = _pack_rows(zt)


_pre = pl.pallas_call(
    _pre_body,
    grid=(E_PAD // BM,),
    in_specs=[
        pl.BlockSpec((BM, D), lambda i: (i, 0)),
        pl.BlockSpec((2 * D, D), lambda i: (0, 0)),
        pl.BlockSpec((D, 1), lambda i: (0, 0)),
    ],
    out_specs=[
        pl.BlockSpec((DP, BM), lambda i: (0, i)),
        pl.BlockSpec((DP, BM), lambda i: (0, i)),
    ],
    out_shape=[
        jax.ShapeDtypeStruct((DP, E_PAD), jnp.int32),
        jax.ShapeDtypeStruct((DP, E_PAD), jnp.int32),
    ],
)


def _post_body(st_ref, x_ref, w2_ref, b2_ref, u1_ref, c1_ref, u2_ref, c2_ref, o_ref):
    st = st_ref[...]
    dn = (((0,), (0,)), ((), ()))
    msgs = lax.dot_general(st, w2_ref[...], dn,
                           preferred_element_type=jnp.float32) * (1.0 / K)
    msgs = msgs + b2_ref[...]
    x = x_ref[...]
    u1 = u1_ref[...]
    h2 = (jnp.dot(x, u1[:D, :], preferred_element_type=jnp.float32)
          + jnp.dot(msgs, u1[D:, :], preferred_element_type=jnp.float32)
          + c1_ref[...])
    h2 = jnp.maximum(h2, 0.0)
    o_ref[...] = jnp.dot(h2, u2_ref[...], preferred_element_type=jnp.float32) + c2_ref[...]


_post = pl.pallas_call(
    _post_body,
    grid=(E_PAD // BM,),
    in_specs=[
        pl.BlockSpec((D, BM), lambda i: (0, i)),
        pl.BlockSpec((BM, D), lambda i: (i, 0)),
        pl.BlockSpec((D, D), lambda i: (0, 0)),
        pl.BlockSpec((1, D), lambda i: (0, 0)),
        pl.BlockSpec((2 * D, D), lambda i: (0, 0)),
        pl.BlockSpec((1, D), lambda i: (0, 0)),
        pl.BlockSpec((D, D), lambda i: (0, 0)),
        pl.BlockSpec((1, D), lambda i: (0, 0)),
    ],
    out_specs=pl.BlockSpec((BM, D), lambda i: (i, 0)),
    out_shape=jax.ShapeDtypeStruct((E_PAD, D), jnp.float32),
)


def _sc_body(adjt_hbm, at_hbm, zt_hbm, st_hbm, zt_v, adj0, adj1, a0, a1,
             slo0, slo1, shi0, shi1, insem0, insem1, outsem0, outsem1):
    cid = lax.axis_index("c")
    sid = lax.axis_index("s")
    ebase = cid * ECS
    prow0 = sid * NPP

    for p in range(NPP):
        pltpu.sync_copy(zt_hbm.at[prow0 + p], zt_v.at[pl.ds(p * E_PAD, E_PAD)])

    def in_slices(n):
        lo = ebase + n * CHE
        return (adjt_hbm.at[:, pl.ds(lo, CHE)],
                at_hbm.at[pl.ds(prow0, NPP), pl.ds(lo, CHE)])

    def fetch(n, adj_v, a_v, sem):
        adjs, ats = in_slices(n)
        pltpu.async_copy(adjs, adj_v, sem)
        pltpu.async_copy(ats, a_v, sem)

    def fetch_wait(n, adj_v, a_v, sem):
        adjs, ats = in_slices(n)
        pltpu.make_async_copy(adjs, adj_v, sem).wait()
        pltpu.make_async_copy(ats, a_v, sem).wait()

    def out_slices(n):
        lo = ebase + n * CHE
        return (st_hbm.at[pl.ds(prow0, NPP), pl.ds(lo, CHE)],
                st_hbm.at[pl.ds(DP + prow0, NPP), pl.ds(lo, CHE)])

    def store(n, s_lo, s_hi, sem):
        olo, ohi = out_slices(n)
        pltpu.async_copy(s_lo, olo, sem)
        pltpu.async_copy(s_hi, ohi, sem)

    def store_wait(n, s_lo, s_hi, sem):
        olo, ohi = out_slices(n)
        pltpu.make_async_copy(s_lo, olo, sem).wait()
        pltpu.make_async_copy(s_hi, ohi, sem).wait()

    coff = [jnp.full((L,), p * E_PAD, jnp.int32) for p in range(NPP)]

    def compute(adj_v, a_v, s_lo, s_hi):
        def group(g, carry):
            av = [plsc.bitcast(a_v[p, pl.ds(g * L, L)], jnp.bfloat16)
                  for p in range(NPP)]
            acc = [None] * NPP
            for k in range(K):
                jv = adj_v[k, pl.ds(g * L, L)]
                for p in range(NPP):
                    zp = plsc.load_gather(zt_v, [jv + coff[p]])
                    zb = plsc.bitcast(zp, jnp.bfloat16)
                    t = jnp.maximum(av[p] + zb, jnp.bfloat16(0))
                    acc[p] = t if k == 0 else acc[p] + t
            for p in range(NPP):
                lo, hi = plsc.unpack(acc[p], format=plsc.PackFormat.INTERLEAVED)
                s_lo[p, pl.ds(g * L, L)] = lo
                s_hi[p, pl.ds(g * L, L)] = hi
            return carry

        lax.fori_loop(0, GRP, group, 0)

    fetch(0, adj0, a0, insem0)
    PAIRS = NCH // 2

    def pair_body(p, carry):
        n0 = p * 2
        fetch(n0 + 1, adj1, a1, insem1)
        fetch_wait(n0, adj0, a0, insem0)

        @pl.when(p > 0)
        def _():
            store_wait(n0 - 2, slo0, shi0, outsem0)
        compute(adj0, a0, slo0, shi0)
        store(n0, slo0, shi0, outsem0)

        @pl.when(p < PAIRS - 1)
        def _():
            fetch(n0 + 2, adj0, a0, insem0)
        fetch_wait(n0 + 1, adj1, a1, insem1)

        @pl.when(p > 0)
        def _():
            store_wait(n0 - 1, slo1, shi1, outsem1)
        compute(adj1, a1, slo1, shi1)
        store(n0 + 1, slo1, shi1, outsem1)
        return carry

    lax.fori_loop(0, PAIRS, pair_body, 0)
    store_wait(NCH - 2, slo0, shi0, outsem0)
    store_wait(NCH - 1, slo1, shi1, outsem1)


@functools.cache
def _sc_gather_mean():
    return pl.kernel(
        _sc_body,
        mesh=plsc.VectorSubcoreMesh(core_axis_name="c", subcore_axis_name="s"),
        compiler_params=pltpu.CompilerParams(needs_layout_passes=False),
        out_type=jax.ShapeDtypeStruct((D, E_PAD), jnp.float32),
        scratch_types=[
            pltpu.VMEM((NPP * E_PAD,), jnp.int32),
            pltpu.VMEM((K, CHE), jnp.int32),
            pltpu.VMEM((K, CHE), jnp.int32),
            pltpu.VMEM((NPP, CHE), jnp.int32),
            pltpu.VMEM((NPP, CHE), jnp.int32),
            pltpu.VMEM((NPP, CHE), jnp.float32),
            pltpu.VMEM((NPP, CHE), jnp.float32),
            pltpu.VMEM((NPP, CHE), jnp.float32),
            pltpu.VMEM((NPP, CHE), jnp.float32),
            pltpu.SemaphoreType.DMA,
            pltpu.SemaphoreType.DMA,
            pltpu.SemaphoreType.DMA,
            pltpu.SemaphoreType.DMA,
        ],
    )


def kernel(edge_features, edge_adjacency, msg_W1, msg_b1, msg_W2, msg_b2,
           upd_W1, upd_b1, upd_W2, upd_b2):
    xp = jnp.zeros((E_PAD, D), jnp.float32).at[:E].set(edge_features)
    adj = jnp.zeros((E_PAD, K), jnp.int32).at[:E].set(edge_adjacency.astype(jnp.int32))
    adjt = adj.T
    at, zt = _pre(xp, msg_W1, msg_b1.reshape(D, 1))
    st = _sc_gather_mean()(adjt, at, zt)
    out = _post(st, xp, msg_W2, msg_b2.reshape(1, D), upd_W1, upd_b1.reshape(1, D),
                upd_W2, upd_b2.reshape(1, D))
    return out[:E]


# speedup vs baseline: 1.1600x; 1.1280x over previous
import functools

import jax
import jax.numpy as jnp
from jax import lax
from jax.experimental import pallas as pl
from jax.experimental.pallas import tpu as pltpu
from jax.experimental.pallas import tpu_sc as plsc

E = 10000
K = 16
D = 128
L = 16
NC, NS = 2, 16
E_PAD = 10240
ECS = E_PAD // NC
DP = D // 2
NPP = DP // NS
CHE = 512
NCH = ECS // CHE
GRP = CHE // L

BM = 5120


def _pack_rows(m):
    lo = lax.bitcast_convert_type(m[:DP].astype(jnp.bfloat16), jnp.uint16)
    hi = lax.bitcast_convert_type(m[DP:].astype(jnp.bfloat16), jnp.uint16)
    word = lo.astype(jnp.uint32) | (hi.astype(jnp.uint32) << 16)
    return lax.bitcast_convert_type(word, jnp.int32)


def _pre_body(x_ref, w_ref, b_ref, at_ref, zt_ref):
    x = x_ref[...]
    w = w_ref[...]
    dn = (((0,), (1,)), ((), ()))
    at = lax.dot_general(w[:D, :], x, dn,
                         preferred_element_type=jnp.float32) + b_ref[...]
    zt = lax.dot_general(w[D:, :], x, dn, preferred_element_type=jnp.float32)
    at_ref[...] = _pack_rows(at)
    zt_ref[...] = _pack_rows(zt)


_pre = pl.pallas_call(
    _pre_body,
    grid=(E_PAD // BM,),
    in_specs=[
        pl.BlockSpec((BM, D), lambda i: (i, 0)),
        pl.BlockSpec((2 * D, D), lambda i: (0, 0)),
        pl.BlockSpec((D, 1), lambda i: (0, 0)),
    ],
    out_specs=[
        pl.BlockSpec((DP, BM), lambda i: (0, i)),
        pl.BlockSpec((DP, BM), lambda i: (0, i)),
    ],
    out_shape=[
        jax.ShapeDtypeStruct((DP, E_PAD), jnp.int32),
        jax.ShapeDtypeStruct((DP, E_PAD), jnp.int32),
    ],
)


def _post_body(st_ref, x_ref, w2_ref, b2_ref, u1_ref, c1_ref, u2_ref, c2_ref, o_ref):
    st = st_ref[...]
    dn = (((0,), (0,)), ((), ()))
    msgs = lax.dot_general(st, w2_ref[...], dn,
                           preferred_element_type=jnp.float32) * (1.0 / K)
    msgs = msgs + b2_ref[...]
    x = x_ref[...]
    u1 = u1_ref[...]
    h2 = (jnp.dot(x, u1[:D, :], preferred_element_type=jnp.float32)
          + jnp.dot(msgs, u1[D:, :], preferred_element_type=jnp.float32)
          + c1_ref[...])
    h2 = jnp.maximum(h2, 0.0)
    o_ref[...] = jnp.dot(h2, u2_ref[...], preferred_element_type=jnp.float32) + c2_ref[...]


_post = pl.pallas_call(
    _post_body,
    grid=(E_PAD // BM,),
    in_specs=[
        pl.BlockSpec((D, BM), lambda i: (0, i)),
        pl.BlockSpec((BM, D), lambda i: (i, 0)),
        pl.BlockSpec((D, D), lambda i: (0, 0)),
        pl.BlockSpec((1, D), lambda i: (0, 0)),
        pl.BlockSpec((2 * D, D), lambda i: (0, 0)),
        pl.BlockSpec((1, D), lambda i: (0, 0)),
        pl.BlockSpec((D, D), lambda i: (0, 0)),
        pl.BlockSpec((1, D), lambda i: (0, 0)),
    ],
    out_specs=pl.BlockSpec((BM, D), lambda i: (i, 0)),
    out_shape=jax.ShapeDtypeStruct((E, D), jnp.float32),
)


def _sc_body(adjt_hbm, at_hbm, zt_hbm, st_hbm, zt_v, adj0, adj1, a0, a1,
             slo0, slo1, shi0, shi1, insem0, insem1, outsem0, outsem1):
    cid = lax.axis_index("c")
    sid = lax.axis_index("s")
    ebase = cid * ECS
    prow0 = sid * NPP

    for p in range(NPP):
        pltpu.sync_copy(zt_hbm.at[prow0 + p], zt_v.at[pl.ds(p * E_PAD, E_PAD)])

    def in_slices(n):
        lo = ebase + n * CHE
        return (adjt_hbm.at[:, pl.ds(lo, CHE)],
                at_hbm.at[pl.ds(prow0, NPP), pl.ds(lo, CHE)])

    def fetch(n, adj_v, a_v, sem):
        adjs, ats = in_slices(n)
        pltpu.async_copy(adjs, adj_v, sem)
        pltpu.async_copy(ats, a_v, sem)

    def fetch_wait(n, adj_v, a_v, sem):
        adjs, ats = in_slices(n)
        pltpu.make_async_copy(adjs, adj_v, sem).wait()
        pltpu.make_async_copy(ats, a_v, sem).wait()

    def out_slices(n):
        lo = ebase + n * CHE
        return (st_hbm.at[pl.ds(prow0, NPP), pl.ds(lo, CHE)],
                st_hbm.at[pl.ds(DP + prow0, NPP), pl.ds(lo, CHE)])

    def store(n, s_lo, s_hi, sem):
        olo, ohi = out_slices(n)
        pltpu.async_copy(s_lo, olo, sem)
        pltpu.async_copy(s_hi, ohi, sem)

    def store_wait(n, s_lo, s_hi, sem):
        olo, ohi = out_slices(n)
        pltpu.make_async_copy(s_lo, olo, sem).wait()
        pltpu.make_async_copy(s_hi, ohi, sem).wait()

    coff = [jnp.full((L,), p * E_PAD, jnp.int32) for p in range(NPP)]

    def compute(adj_v, a_v, s_lo, s_hi):
        def group(g, carry):
            av = [plsc.bitcast(a_v[p, pl.ds(g * L, L)], jnp.bfloat16)
                  for p in range(NPP)]
            acc = [None] * NPP
            for k in range(K):
                jv = adj_v[k, pl.ds(g * L, L)]
                for p in range(NPP):
                    zp = plsc.load_gather(zt_v, [jv + coff[p]])
                    zb = plsc.bitcast(zp, jnp.bfloat16)
                    t = jnp.maximum(av[p] + zb, jnp.bfloat16(0))
                    acc[p] = t if k == 0 else acc[p] + t
            for p in range(NPP):
                lo, hi = plsc.unpack(acc[p], format=plsc.PackFormat.INTERLEAVED)
                s_lo[p, pl.ds(g * L, L)] = lo
                s_hi[p, pl.ds(g * L, L)] = hi
            return carry

        lax.fori_loop(0, GRP, group, 0)

    fetch(0, adj0, a0, insem0)
    PAIRS = NCH // 2

    def pair_body(p, carry):
        n0 = p * 2
        fetch(n0 + 1, adj1, a1, insem1)
        fetch_wait(n0, adj0, a0, insem0)

        @pl.when(p > 0)
        def _():
            store_wait(n0 - 2, slo0, shi0, outsem0)
        compute(adj0, a0, slo0, shi0)
        store(n0, slo0, shi0, outsem0)

        @pl.when(p < PAIRS - 1)
        def _():
            fetch(n0 + 2, adj0, a0, insem0)
        fetch_wait(n0 + 1, adj1, a1, insem1)

        @pl.when(p > 0)
        def _():
            store_wait(n0 - 1, slo1, shi1, outsem1)
        compute(adj1, a1, slo1, shi1)
        store(n0 + 1, slo1, shi1, outsem1)
        return carry

    lax.fori_loop(0, PAIRS, pair_body, 0)
    store_wait(NCH - 2, slo0, shi0, outsem0)
    store_wait(NCH - 1, slo1, shi1, outsem1)


@functools.cache
def _sc_gather_mean():
    return pl.kernel(
        _sc_body,
        mesh=plsc.VectorSubcoreMesh(core_axis_name="c", subcore_axis_name="s"),
        compiler_params=pltpu.CompilerParams(needs_layout_passes=False),
        out_type=jax.ShapeDtypeStruct((D, E_PAD), jnp.float32),
        scratch_types=[
            pltpu.VMEM((NPP * E_PAD,), jnp.int32),
            pltpu.VMEM((K, CHE), jnp.int32),
            pltpu.VMEM((K, CHE), jnp.int32),
            pltpu.VMEM((NPP, CHE), jnp.int32),
            pltpu.VMEM((NPP, CHE), jnp.int32),
            pltpu.VMEM((NPP, CHE), jnp.float32),
            pltpu.VMEM((NPP, CHE), jnp.float32),
            pltpu.VMEM((NPP, CHE), jnp.float32),
            pltpu.VMEM((NPP, CHE), jnp.float32),
            pltpu.SemaphoreType.DMA,
            pltpu.SemaphoreType.DMA,
            pltpu.SemaphoreType.DMA,
            pltpu.SemaphoreType.DMA,
        ],
    )


def kernel(edge_features, edge_adjacency, msg_W1, msg_b1, msg_W2, msg_b2,
           upd_W1, upd_b1, upd_W2, upd_b2):
    adj = jnp.zeros((E_PAD, K), jnp.int32).at[:E].set(edge_adjacency.astype(jnp.int32))
    adjt = adj.T
    at, zt = _pre(edge_features, msg_W1, msg_b1.reshape(D, 1))
    st = _sc_gather_mean()(adjt, at, zt)
    out = _post(st, edge_features, msg_W2, msg_b2.reshape(1, D),
                upd_W1, upd_b1.reshape(1, D), upd_W2, upd_b2.reshape(1, D))
    return out
